# baseline (speedup 1.0000x reference)
import functools

import jax
import jax.numpy as jnp
from jax import lax
from jax.experimental import pallas as pl
from jax.experimental.pallas import tpu as pltpu
from jax.experimental.pallas import tpu_sc as plsc

N_NODES = 100000
N_EDGES = 3200000
N_GRAPHS = 16
MAX_STEPS = 8
N_STARTS = 64
N_ANSWERS = 160

NC, NS, L = 2, 16, 16
NW = NC * NS
TBL_W = 3200
EC = 6400
N_CHUNKS = N_EDGES // EC
CPW_MAX = -(-N_CHUNKS // NW)
GU = 5
NODES_PW = 3072
N_NODES_PAD = NW * NODES_PW + 2048


def _sc_body(edge_hbm, lab_hbm, sl_hbm, al_hbm,
             mask_hbm, gated_hbm, nstart_hbm, nans_hbm,
             start_tbl, ans_tbl, sl_v, al_v,
             ei_v, lab_v, gated_v, mask_v, nb_v,
             in_sem, out_sem):
    wid = lax.axis_index("s") * NC + lax.axis_index("c")
    iota = lax.broadcasted_iota(jnp.int32, (L,), 0)

    zeros16 = jnp.zeros((L,), jnp.int32)

    def _zero(i, _):
        start_tbl[pl.ds(pl.multiple_of(i * L, 8), L)] = zeros16
        ans_tbl[pl.ds(pl.multiple_of(i * L, 8), L)] = zeros16
        return 0

    lax.fori_loop(0, TBL_W // L, _zero, 0)

    pltpu.sync_copy(sl_hbm, sl_v)
    pltpu.sync_copy(al_hbm, al_v)

    lane0 = iota == 0

    def _set_bits(n, tbl, buf):
        def _one(i, _):
            idx = plsc.load_gather(buf, [jnp.full((L,), i, jnp.int32)])
            word = idx >> 5
            w = plsc.load_gather(tbl, [word])
            plsc.store_scatter(tbl, [word], w | (1 << (idx & 31)), mask=lane0)
            return 0
        lax.fori_loop(0, n, _one, 0)

    _set_bits(N_STARTS, start_tbl, sl_v)
    _set_bits(N_ANSWERS, ans_tbl, al_v)

    def _bit(tbl, n):
        w = plsc.load_gather(tbl, [n >> 5])
        return (w >> (n & 31)) & 1

    nb_stride = nb_v.shape[0] // 2

    def _node_range(node_base, n_groups, dma_words):
        def _node_group(g, _):
            n = node_base + g * L + iota
            off = pl.multiple_of(g * L, 8)
            nb_v[pl.ds(off, L)] = _bit(start_tbl, n)
            nb_v[pl.ds(nb_stride + off, L)] = _bit(ans_tbl, n)
            return 0

        lax.fori_loop(0, n_groups, _node_group, 0)
        hoff = pl.multiple_of(node_base, 8)
        pltpu.sync_copy(nb_v.at[pl.ds(0, dma_words)],
                        nstart_hbm.at[pl.ds(hoff, dma_words)])
        pltpu.sync_copy(nb_v.at[pl.ds(nb_stride, dma_words)],
                        nans_hbm.at[pl.ds(hoff, dma_words)])

    _node_range(wid * NODES_PW, NODES_PW // L, NODES_PW)
    tail_nodes = N_NODES_PAD - NW * NODES_PW
    if tail_nodes:
        @pl.when(wid == NW - 1)
        def _():
            _node_range(NW * NODES_PW, tail_nodes // L, tail_nodes)

    n_my = jnp.where(wid < (N_CHUNKS % NW), CPW_MAX, CPW_MAX - 1) \
        if N_CHUNKS % NW else CPW_MAX

    def _base(t):
        return pl.multiple_of((wid + t * NW) * EC, 128)

    def _issue_in(t, b):
        base = _base(t)
        pltpu.async_copy(edge_hbm.at[pl.ds(0, 2), pl.ds(base, EC)],
                         ei_v.at[b], in_sem.at[b])
        pltpu.async_copy(lab_hbm.at[pl.ds(base, EC)], lab_v.at[b],
                         in_sem.at[b])

    def _wait_in(t, b):
        pltpu.make_async_copy(edge_hbm.at[pl.ds(0, 2), pl.ds(_base(t), EC)],
                              ei_v.at[b], in_sem.at[b]).wait()
        pltpu.make_async_copy(lab_hbm.at[pl.ds(_base(t), EC)], lab_v.at[b],
                              in_sem.at[b]).wait()

    def _issue_out(t, b):
        base = _base(t)
        pltpu.async_copy(mask_v.at[b], mask_hbm.at[pl.ds(base, EC)],
                         out_sem.at[b])
        pltpu.async_copy(gated_v.at[b], gated_hbm.at[pl.ds(base, EC)],
                         out_sem.at[b])

    def _wait_out(t, b):
        base = _base(t)
        pltpu.make_async_copy(mask_v.at[b], mask_hbm.at[pl.ds(base, EC)],
                              out_sem.at[b]).wait()
        pltpu.make_async_copy(gated_v.at[b], gated_hbm.at[pl.ds(base, EC)],
                              out_sem.at[b]).wait()

    def _compute(b):
        def _group(g, _):
            for k in range(GU):
                off = pl.multiple_of(g * (GU * L) + k * L, 8)
                u = ei_v[b, 0, pl.ds(off, L)]
                v = ei_v[b, 1, pl.ds(off, L)]
                m = _bit(start_tbl, u) | _bit(start_tbl, v)
                labv = lab_v[b, pl.ds(off, L)]
                mask_v[b, pl.ds(off, L)] = m
                gated_v[b, pl.ds(off, L)] = jnp.where(
                    m == 1, labv, jnp.zeros((L,), jnp.float32))
            return 0

        lax.fori_loop(0, EC // (GU * L), _group, 0)

    def _guarded(t, fn):
        if t < CPW_MAX - 1:
            fn()
        else:
            pl.when(t < n_my)(fn)

    for t in range(CPW_MAX):
        b = t % 2

        def _phase(t=t, b=b):
            base = _base(t)
            pltpu.sync_copy(edge_hbm.at[0, pl.ds(base, EC)], ei_v.at[b, 0])
            pltpu.sync_copy(edge_hbm.at[1, pl.ds(base, EC)], ei_v.at[b, 1])
            pltpu.sync_copy(lab_hbm.at[pl.ds(base, EC)], lab_v.at[b])
            _compute(b)
            pltpu.sync_copy(mask_v.at[b], mask_hbm.at[pl.ds(base, EC)])
            pltpu.sync_copy(gated_v.at[b], gated_hbm.at[pl.ds(base, EC)])

        _guarded(t, _phase)


@jax.jit
def _sc_call(edge_index, labels, start_locals, answer_locals):
    mesh = plsc.VectorSubcoreMesh(core_axis_name="c", subcore_axis_name="s",
                                  num_cores=NC, num_subcores=NS)
    out_type = (
        jax.ShapeDtypeStruct((N_EDGES,), jnp.int32),
        jax.ShapeDtypeStruct((N_EDGES,), jnp.float32),
        jax.ShapeDtypeStruct((N_NODES_PAD,), jnp.int32),
        jax.ShapeDtypeStruct((N_NODES_PAD,), jnp.int32),
    )
    scratch = [
        pltpu.VMEM((TBL_W,), jnp.int32),
        pltpu.VMEM((TBL_W,), jnp.int32),
        pltpu.VMEM((N_STARTS,), jnp.int32),
        pltpu.VMEM((N_ANSWERS,), jnp.int32),
        pltpu.VMEM((2, 2, EC), jnp.int32),
        pltpu.VMEM((2, EC), jnp.float32),
        pltpu.VMEM((2, EC), jnp.float32),
        pltpu.VMEM((2, EC), jnp.int32),
        pltpu.VMEM((2 * NODES_PW,), jnp.int32),
        pltpu.SemaphoreType.DMA((2,)),
        pltpu.SemaphoreType.DMA((2,)),
    ]
    params = pltpu.CompilerParams(needs_layout_passes=False)
    return pl.kernel(_sc_body, out_type=out_type, mesh=mesh,
                     scratch_types=scratch,
                     compiler_params=params)(edge_index, labels,
                                             start_locals, answer_locals)


def kernel(edge_index, edge_batch, node_global_ids, node_ptr, edge_ptr,
           start_node_locals, start_ptr, start_entity_ids, start_entity_ptr,
           answer_node_locals, answer_ptr, answer_entity_ids, edge_relations,
           edge_labels, top_edge_mask, gt_path_edge_local_ids, gt_edge_ptr,
           gt_path_exists, is_answer_reachable, bypass_action_mask):
    num_graphs = node_ptr.shape[0] - 1

    mask_i, gated_labels, ns_i, na_i = _sc_call(
        edge_index, edge_labels, start_node_locals, answer_node_locals)

    edge_starts_mask = mask_i.astype(bool)
    node_is_start = ns_i[:N_NODES].astype(bool)
    node_is_answer = na_i[:N_NODES].astype(bool)
    visited_nodes = node_is_start

    selected_mask = jnp.zeros((N_EDGES,), dtype=bool)
    selection_order = jnp.full((N_EDGES,), -1, dtype=jnp.int32)
    current_tail = jnp.full((num_graphs,), -1, dtype=jnp.int32)
    prev_tail = jnp.full((num_graphs,), -1, dtype=jnp.int32)
    done = jnp.zeros((num_graphs,), dtype=bool)
    step_counts = jnp.zeros((num_graphs,), dtype=jnp.int32)
    actions = jnp.full((num_graphs, MAX_STEPS + 1), -1, dtype=jnp.int32)
    answer_hits = jnp.zeros((num_graphs,), dtype=bool)
    start_counts = start_ptr[1:] - start_ptr[:-1]

    return (edge_starts_mask, node_is_start, node_is_answer, visited_nodes,
            selected_mask, selection_order, current_tail, prev_tail, done,
            step_counts, actions, answer_hits, start_counts, gated_labels)


# speedup vs baseline: 1.3388x; 1.3388x over previous
import functools

import jax
import jax.numpy as jnp
from jax import lax
from jax.experimental import pallas as pl
from jax.experimental.pallas import tpu as pltpu
from jax.experimental.pallas import tpu_sc as plsc

N_NODES = 100000
N_EDGES = 3200000
N_GRAPHS = 16
MAX_STEPS = 8
N_STARTS = 64
N_ANSWERS = 160

NC, NS, L = 2, 16, 16
NW = NC * NS
TBL_W = 3200
EC = 6400
N_CHUNKS = N_EDGES // EC
CPW_MAX = -(-N_CHUNKS // NW)
GU = 1
NODES_PW = 3072
N_NODES_PAD = NW * NODES_PW + 2048


def _sc_body(edge_hbm, lab_hbm, sl_hbm, al_hbm,
             mask_hbm, gated_hbm, nstart_hbm, nans_hbm,
             start_tbl, ans_tbl, sl_v, al_v,
             ei_v, lab_v, gated_v, mask_v, nb_v,
             in_sem, out_sem):
    wid = lax.axis_index("s") * NC + lax.axis_index("c")
    iota = lax.broadcasted_iota(jnp.int32, (L,), 0)

    zeros16 = jnp.zeros((L,), jnp.int32)

    def _zero(i, _):
        start_tbl[pl.ds(pl.multiple_of(i * L, 8), L)] = zeros16
        ans_tbl[pl.ds(pl.multiple_of(i * L, 8), L)] = zeros16
        return 0

    lax.fori_loop(0, TBL_W // L, _zero, 0)

    pltpu.sync_copy(sl_hbm, sl_v)
    pltpu.sync_copy(al_hbm, al_v)

    lane0 = iota == 0

    def _set_bits(n, tbl, buf):
        def _one(i, _):
            idx = plsc.load_gather(buf, [jnp.full((L,), i, jnp.int32)])
            word = idx >> 5
            w = plsc.load_gather(tbl, [word])
            plsc.store_scatter(tbl, [word], w | (1 << (idx & 31)), mask=lane0)
            return 0
        lax.fori_loop(0, n, _one, 0)

    _set_bits(N_STARTS, start_tbl, sl_v)
    _set_bits(N_ANSWERS, ans_tbl, al_v)

    def _bit(tbl, n):
        w = plsc.load_gather(tbl, [n >> 5])
        return (w >> (n & 31)) & 1

    nb_stride = nb_v.shape[0] // 2

    def _node_range(node_base, n_groups, dma_words):
        def _node_group(g, _):
            n = node_base + g * L + iota
            off = pl.multiple_of(g * L, 8)
            nb_v[pl.ds(off, L)] = _bit(start_tbl, n)
            nb_v[pl.ds(nb_stride + off, L)] = _bit(ans_tbl, n)
            return 0

        lax.fori_loop(0, n_groups, _node_group, 0)
        hoff = pl.multiple_of(node_base, 8)
        pltpu.sync_copy(nb_v.at[pl.ds(0, dma_words)],
                        nstart_hbm.at[pl.ds(hoff, dma_words)])
        pltpu.sync_copy(nb_v.at[pl.ds(nb_stride, dma_words)],
                        nans_hbm.at[pl.ds(hoff, dma_words)])

    _node_range(wid * NODES_PW, NODES_PW // L, NODES_PW)
    tail_nodes = N_NODES_PAD - NW * NODES_PW
    if tail_nodes:
        @pl.when(wid == NW - 1)
        def _():
            _node_range(NW * NODES_PW, tail_nodes // L, tail_nodes)

    n_my = jnp.where(wid < (N_CHUNKS % NW), CPW_MAX, CPW_MAX - 1) \
        if N_CHUNKS % NW else CPW_MAX

    def _base(t):
        return pl.multiple_of((wid + t * NW) * EC, 128)

    def _issue_in(t, b):
        base = _base(t)
        pltpu.async_copy(edge_hbm.at[pl.ds(0, 2), pl.ds(base, EC)],
                         ei_v.at[b], in_sem.at[b])
        pltpu.async_copy(lab_hbm.at[pl.ds(base, EC)], lab_v.at[b],
                         in_sem.at[b])

    def _wait_in(t, b):
        pltpu.make_async_copy(edge_hbm.at[pl.ds(0, 2), pl.ds(_base(t), EC)],
                              ei_v.at[b], in_sem.at[b]).wait()
        pltpu.make_async_copy(lab_hbm.at[pl.ds(_base(t), EC)], lab_v.at[b],
                              in_sem.at[b]).wait()

    def _issue_out(t, b):
        base = _base(t)
        pltpu.async_copy(mask_v.at[b], mask_hbm.at[pl.ds(base, EC)],
                         out_sem.at[b])
        pltpu.async_copy(gated_v.at[b], gated_hbm.at[pl.ds(base, EC)],
                         out_sem.at[b])

    def _wait_out(t, b):
        base = _base(t)
        pltpu.make_async_copy(mask_v.at[b], mask_hbm.at[pl.ds(base, EC)],
                              out_sem.at[b]).wait()
        pltpu.make_async_copy(gated_v.at[b], gated_hbm.at[pl.ds(base, EC)],
                              out_sem.at[b]).wait()

    def _compute(b):
        def _group(g, _):
            for k in range(GU):
                off = pl.multiple_of(g * (GU * L) + k * L, 8)
                u = ei_v[b, 0, pl.ds(off, L)]
                v = ei_v[b, 1, pl.ds(off, L)]
                m = _bit(start_tbl, u) | _bit(start_tbl, v)
                labv = lab_v[b, pl.ds(off, L)]
                mask_v[b, pl.ds(off, L)] = m
                gated_v[b, pl.ds(off, L)] = jnp.where(
                    m == 1, labv, jnp.zeros((L,), jnp.float32))
            return 0

        lax.fori_loop(0, EC // (GU * L), _group, 0)

    def _guarded(t, fn):
        if t < CPW_MAX - 1:
            fn()
        else:
            pl.when(t < n_my)(fn)

    for t in range(CPW_MAX):
        b = t % 2

        def _phase(t=t, b=b):
            base = _base(t)
            pltpu.sync_copy(edge_hbm.at[pl.ds(0, 2), pl.ds(base, EC)],
                            ei_v.at[b])
            pltpu.sync_copy(lab_hbm.at[pl.ds(base, EC)], lab_v.at[b])
            _compute(b)
            pltpu.sync_copy(mask_v.at[b], mask_hbm.at[pl.ds(base, EC)])
            pltpu.sync_copy(gated_v.at[b], gated_hbm.at[pl.ds(base, EC)])

        _guarded(t, _phase)


@jax.jit
def _sc_call(edge_index, labels, start_locals, answer_locals):
    mesh = plsc.VectorSubcoreMesh(core_axis_name="c", subcore_axis_name="s",
                                  num_cores=NC, num_subcores=NS)
    out_type = (
        jax.ShapeDtypeStruct((N_EDGES,), jnp.int32),
        jax.ShapeDtypeStruct((N_EDGES,), jnp.float32),
        jax.ShapeDtypeStruct((N_NODES_PAD,), jnp.int32),
        jax.ShapeDtypeStruct((N_NODES_PAD,), jnp.int32),
    )
    scratch = [
        pltpu.VMEM((TBL_W,), jnp.int32),
        pltpu.VMEM((TBL_W,), jnp.int32),
        pltpu.VMEM((N_STARTS,), jnp.int32),
        pltpu.VMEM((N_ANSWERS,), jnp.int32),
        pltpu.VMEM((2, 2, EC), jnp.int32),
        pltpu.VMEM((2, EC), jnp.float32),
        pltpu.VMEM((2, EC), jnp.float32),
        pltpu.VMEM((2, EC), jnp.int32),
        pltpu.VMEM((2 * NODES_PW,), jnp.int32),
        pltpu.SemaphoreType.DMA((2,)),
        pltpu.SemaphoreType.DMA((2,)),
    ]
    params = pltpu.CompilerParams(needs_layout_passes=False)
    return pl.kernel(_sc_body, out_type=out_type, mesh=mesh,
                     scratch_types=scratch,
                     compiler_params=params)(edge_index, labels,
                                             start_locals, answer_locals)


def kernel(edge_index, edge_batch, node_global_ids, node_ptr, edge_ptr,
           start_node_locals, start_ptr, start_entity_ids, start_entity_ptr,
           answer_node_locals, answer_ptr, answer_entity_ids, edge_relations,
           edge_labels, top_edge_mask, gt_path_edge_local_ids, gt_edge_ptr,
           gt_path_exists, is_answer_reachable, bypass_action_mask):
    num_graphs = node_ptr.shape[0] - 1

    mask_i, gated_labels, ns_i, na_i = _sc_call(
        edge_index, edge_labels, start_node_locals, answer_node_locals)

    edge_starts_mask = mask_i.astype(bool)
    node_is_start = ns_i[:N_NODES].astype(bool)
    node_is_answer = na_i[:N_NODES].astype(bool)
    visited_nodes = node_is_start

    selected_mask = jnp.zeros((N_EDGES,), dtype=bool)
    selection_order = jnp.full((N_EDGES,), -1, dtype=jnp.int32)
    current_tail = jnp.full((num_graphs,), -1, dtype=jnp.int32)
    prev_tail = jnp.full((num_graphs,), -1, dtype=jnp.int32)
    done = jnp.zeros((num_graphs,), dtype=bool)
    step_counts = jnp.zeros((num_graphs,), dtype=jnp.int32)
    actions = jnp.full((num_graphs, MAX_STEPS + 1), -1, dtype=jnp.int32)
    answer_hits = jnp.zeros((num_graphs,), dtype=bool)
    start_counts = start_ptr[1:] - start_ptr[:-1]

    return (edge_starts_mask, node_is_start, node_is_answer, visited_nodes,
            selected_mask, selection_order, current_tail, prev_tail, done,
            step_counts, actions, answer_hits, start_counts, gated_labels)


# speedup vs baseline: 1.6361x; 1.2221x over previous
import functools

import jax
import jax.numpy as jnp
from jax import lax
from jax.experimental import pallas as pl
from jax.experimental.pallas import tpu as pltpu
from jax.experimental.pallas import tpu_sc as plsc

N_NODES = 100000
N_EDGES = 3200000
N_GRAPHS = 16
MAX_STEPS = 8
N_STARTS = 64
N_ANSWERS = 160

NC, NS, L = 2, 16, 16
NW = NC * NS
TBL_W = 3200
EC = 6400
N_CHUNKS = N_EDGES // EC
CPW_MAX = -(-N_CHUNKS // NW)
GU = 1
NODES_PW = 3072
N_NODES_PAD = NW * NODES_PW + 2048


def _sc_body(edge_hbm, lab_hbm, sl_hbm, al_hbm,
             mask_hbm, gated_hbm, nstart_hbm, nans_hbm,
             start_tbl, ans_tbl, sl_v, al_v,
             ei_v, lab_v, gated_v, mask_v, nb_v,
             in_sem, out_sem):
    wid = lax.axis_index("s") * NC + lax.axis_index("c")
    iota = lax.broadcasted_iota(jnp.int32, (L,), 0)

    zeros16 = jnp.zeros((L,), jnp.int32)

    def _zero(i, _):
        start_tbl[pl.ds(pl.multiple_of(i * L, 8), L)] = zeros16
        ans_tbl[pl.ds(pl.multiple_of(i * L, 8), L)] = zeros16
        return 0

    lax.fori_loop(0, TBL_W // L, _zero, 0)

    pltpu.sync_copy(sl_hbm, sl_v)
    pltpu.sync_copy(al_hbm, al_v)

    lane0 = iota == 0

    def _set_bits(n, tbl, buf):
        def _one(i, _):
            idx = plsc.load_gather(buf, [jnp.full((L,), i, jnp.int32)])
            word = idx >> 5
            w = plsc.load_gather(tbl, [word])
            plsc.store_scatter(tbl, [word], w | (1 << (idx & 31)), mask=lane0)
            return 0
        lax.fori_loop(0, n, _one, 0)

    _set_bits(N_STARTS, start_tbl, sl_v)
    _set_bits(N_ANSWERS, ans_tbl, al_v)

    def _bit(tbl, n):
        w = plsc.load_gather(tbl, [n >> 5])
        return (w >> (n & 31)) & 1

    nb_stride = nb_v.shape[0] // 2

    def _node_range(node_base, n_groups, dma_words):
        def _node_group(g, _):
            n = node_base + g * L + iota
            off = pl.multiple_of(g * L, 8)
            nb_v[pl.ds(off, L)] = _bit(start_tbl, n)
            nb_v[pl.ds(nb_stride + off, L)] = _bit(ans_tbl, n)
            return 0

        lax.fori_loop(0, n_groups, _node_group, 0)
        hoff = pl.multiple_of(node_base, 8)
        pltpu.sync_copy(nb_v.at[pl.ds(0, dma_words)],
                        nstart_hbm.at[pl.ds(hoff, dma_words)])
        pltpu.sync_copy(nb_v.at[pl.ds(nb_stride, dma_words)],
                        nans_hbm.at[pl.ds(hoff, dma_words)])

    _node_range(wid * NODES_PW, NODES_PW // L, NODES_PW)
    tail_nodes = N_NODES_PAD - NW * NODES_PW
    if tail_nodes:
        @pl.when(wid == NW - 1)
        def _():
            _node_range(NW * NODES_PW, tail_nodes // L, tail_nodes)

    n_my = jnp.where(wid < (N_CHUNKS % NW), CPW_MAX, CPW_MAX - 1) \
        if N_CHUNKS % NW else CPW_MAX

    def _base(t):
        return pl.multiple_of((wid + t * NW) * EC, 128)

    def _issue_in(t, b):
        base = _base(t)
        pltpu.async_copy(edge_hbm.at[pl.ds(0, 2), pl.ds(base, EC)],
                         ei_v.at[b], in_sem.at[b])
        pltpu.async_copy(lab_hbm.at[pl.ds(base, EC)], lab_v.at[b],
                         in_sem.at[b])

    def _wait_in(t, b):
        pltpu.make_async_copy(edge_hbm.at[pl.ds(0, 2), pl.ds(_base(t), EC)],
                              ei_v.at[b], in_sem.at[b]).wait()
        pltpu.make_async_copy(lab_hbm.at[pl.ds(_base(t), EC)], lab_v.at[b],
                              in_sem.at[b]).wait()

    def _issue_out(t, b):
        base = _base(t)
        pltpu.async_copy(mask_v.at[b], mask_hbm.at[pl.ds(base, EC)],
                         out_sem.at[b])
        pltpu.async_copy(gated_v.at[b], gated_hbm.at[pl.ds(base, EC)],
                         out_sem.at[b])

    def _wait_out(t, b):
        base = _base(t)
        pltpu.make_async_copy(mask_v.at[b], mask_hbm.at[pl.ds(base, EC)],
                              out_sem.at[b]).wait()
        pltpu.make_async_copy(gated_v.at[b], gated_hbm.at[pl.ds(base, EC)],
                              out_sem.at[b]).wait()

    def _compute(b):
        def _group(g, _):
            for k in range(GU):
                off = pl.multiple_of(g * (GU * L) + k * L, 8)
                u = ei_v[b, 0, pl.ds(off, L)]
                v = ei_v[b, 1, pl.ds(off, L)]
                m = _bit(start_tbl, u) | _bit(start_tbl, v)
                labv = lab_v[b, pl.ds(off, L)]
                mask_v[b, pl.ds(off, L)] = m
                gated_v[b, pl.ds(off, L)] = jnp.where(
                    m == 1, labv, jnp.zeros((L,), jnp.float32))
            return 0

        lax.fori_loop(0, EC // (GU * L), _group, 0)

    def _guarded(t, fn):
        if t < CPW_MAX - 1:
            fn()
        else:
            pl.when(t < n_my)(fn)

    _guarded(0, lambda: _issue_in(0, 0))
    if CPW_MAX > 1:
        _guarded(1, lambda: _issue_in(1, 1))
    for t in range(CPW_MAX):
        b = t % 2

        def _phase(t=t, b=b):
            _wait_in(t, b)
            if t >= 2:
                _wait_out(t - 2, b)
            _compute(b)
            _issue_out(t, b)
            if t + 2 < CPW_MAX:
                _guarded(t + 2, lambda: _issue_in(t + 2, b))

        _guarded(t, _phase)
    for t in range(max(0, CPW_MAX - 3), CPW_MAX):
        def _drain(t=t):
            _wait_out(t, t % 2)
        pl.when(jnp.logical_and(t < n_my, t >= n_my - 2))(_drain)


@jax.jit
def _sc_call(edge_index, labels, start_locals, answer_locals):
    mesh = plsc.VectorSubcoreMesh(core_axis_name="c", subcore_axis_name="s",
                                  num_cores=NC, num_subcores=NS)
    out_type = (
        jax.ShapeDtypeStruct((N_EDGES,), jnp.int32),
        jax.ShapeDtypeStruct((N_EDGES,), jnp.float32),
        jax.ShapeDtypeStruct((N_NODES_PAD,), jnp.int32),
        jax.ShapeDtypeStruct((N_NODES_PAD,), jnp.int32),
    )
    scratch = [
        pltpu.VMEM((TBL_W,), jnp.int32),
        pltpu.VMEM((TBL_W,), jnp.int32),
        pltpu.VMEM((N_STARTS,), jnp.int32),
        pltpu.VMEM((N_ANSWERS,), jnp.int32),
        pltpu.VMEM((2, 2, EC), jnp.int32),
        pltpu.VMEM((2, EC), jnp.float32),
        pltpu.VMEM((2, EC), jnp.float32),
        pltpu.VMEM((2, EC), jnp.int32),
        pltpu.VMEM((2 * NODES_PW,), jnp.int32),
        pltpu.SemaphoreType.DMA((2,)),
        pltpu.SemaphoreType.DMA((2,)),
    ]
    params = pltpu.CompilerParams(needs_layout_passes=False)
    return pl.kernel(_sc_body, out_type=out_type, mesh=mesh,
                     scratch_types=scratch,
                     compiler_params=params)(edge_index, labels,
                                             start_locals, answer_locals)


def kernel(edge_index, edge_batch, node_global_ids, node_ptr, edge_ptr,
           start_node_locals, start_ptr, start_entity_ids, start_entity_ptr,
           answer_node_locals, answer_ptr, answer_entity_ids, edge_relations,
           edge_labels, top_edge_mask, gt_path_edge_local_ids, gt_edge_ptr,
           gt_path_exists, is_answer_reachable, bypass_action_mask):
    num_graphs = node_ptr.shape[0] - 1

    mask_i, gated_labels, ns_i, na_i = _sc_call(
        edge_index, edge_labels, start_node_locals, answer_node_locals)

    edge_starts_mask = mask_i.astype(bool)
    node_is_start = ns_i[:N_NODES].astype(bool)
    node_is_answer = na_i[:N_NODES].astype(bool)
    visited_nodes = node_is_start

    selected_mask = jnp.zeros((N_EDGES,), dtype=bool)
    selection_order = jnp.full((N_EDGES,), -1, dtype=jnp.int32)
    current_tail = jnp.full((num_graphs,), -1, dtype=jnp.int32)
    prev_tail = jnp.full((num_graphs,), -1, dtype=jnp.int32)
    done = jnp.zeros((num_graphs,), dtype=bool)
    step_counts = jnp.zeros((num_graphs,), dtype=jnp.int32)
    actions = jnp.full((num_graphs, MAX_STEPS + 1), -1, dtype=jnp.int32)
    answer_hits = jnp.zeros((num_graphs,), dtype=bool)
    start_counts = start_ptr[1:] - start_ptr[:-1]

    return (edge_starts_mask, node_is_start, node_is_answer, visited_nodes,
            selected_mask, selection_order, current_tail, prev_tail, done,
            step_counts, actions, answer_hits, start_counts, gated_labels)


# speedup vs baseline: 2.1374x; 1.3064x over previous
import functools

import jax
import jax.numpy as jnp
from jax import lax
from jax.experimental import pallas as pl
from jax.experimental.pallas import tpu as pltpu
from jax.experimental.pallas import tpu_sc as plsc

N_NODES = 100000
N_EDGES = 3200000
N_GRAPHS = 16
MAX_STEPS = 8
N_STARTS = 64
N_ANSWERS = 160

NC, NS, L = 2, 16, 16
NW = NC * NS
TBL_W = 3200
EC = 6400
N_CHUNKS = N_EDGES // EC
CPW_MAX = -(-N_CHUNKS // NW)
GU = 4
NODES_PW = 3072
N_NODES_PAD = NW * NODES_PW + 2048


def _sc_body(edge_hbm, lab_hbm, sl_hbm, al_hbm,
             mask_hbm, gated_hbm, nstart_hbm, nans_hbm,
             start_tbl, ans_tbl, sl_v, al_v,
             ei_v, lab_v, gated_v, mask_v, nb_v,
             in_sem, out_sem):
    wid = lax.axis_index("s") * NC + lax.axis_index("c")
    iota = lax.broadcasted_iota(jnp.int32, (L,), 0)

    zeros16 = jnp.zeros((L,), jnp.int32)

    def _zero(i, _):
        start_tbl[pl.ds(pl.multiple_of(i * L, 8), L)] = zeros16
        ans_tbl[pl.ds(pl.multiple_of(i * L, 8), L)] = zeros16
        return 0

    lax.fori_loop(0, TBL_W // L, _zero, 0)

    pltpu.sync_copy(sl_hbm, sl_v)
    pltpu.sync_copy(al_hbm, al_v)

    lane0 = iota == 0

    def _set_bits(n, tbl, buf):
        def _one(i, _):
            idx = plsc.load_gather(buf, [jnp.full((L,), i, jnp.int32)])
            word = idx >> 5
            w = plsc.load_gather(tbl, [word])
            plsc.store_scatter(tbl, [word], w | (1 << (idx & 31)), mask=lane0)
            return 0
        lax.fori_loop(0, n, _one, 0)

    _set_bits(N_STARTS, start_tbl, sl_v)
    _set_bits(N_ANSWERS, ans_tbl, al_v)

    def _bit(tbl, n):
        w = plsc.load_gather(tbl, [n >> 5])
        return (w >> (n & 31)) & 1

    nb_stride = nb_v.shape[0] // 2

    def _node_range(node_base, n_groups, dma_words):
        def _node_group(g, _):
            n = node_base + g * L + iota
            off = pl.multiple_of(g * L, 8)
            nb_v[pl.ds(off, L)] = _bit(start_tbl, n)
            nb_v[pl.ds(nb_stride + off, L)] = _bit(ans_tbl, n)
            return 0

        lax.fori_loop(0, n_groups, _node_group, 0)
        hoff = pl.multiple_of(node_base, 8)
        pltpu.sync_copy(nb_v.at[pl.ds(0, dma_words)],
                        nstart_hbm.at[pl.ds(hoff, dma_words)])
        pltpu.sync_copy(nb_v.at[pl.ds(nb_stride, dma_words)],
                        nans_hbm.at[pl.ds(hoff, dma_words)])

    _node_range(wid * NODES_PW, NODES_PW // L, NODES_PW)
    tail_nodes = N_NODES_PAD - NW * NODES_PW
    if tail_nodes:
        @pl.when(wid == NW - 1)
        def _():
            _node_range(NW * NODES_PW, tail_nodes // L, tail_nodes)

    n_my = jnp.where(wid < (N_CHUNKS % NW), CPW_MAX, CPW_MAX - 1) \
        if N_CHUNKS % NW else CPW_MAX

    def _base(t):
        return pl.multiple_of((wid + t * NW) * EC, 128)

    def _issue_in(t, b):
        base = _base(t)
        pltpu.async_copy(edge_hbm.at[pl.ds(0, 2), pl.ds(base, EC)],
                         ei_v.at[b], in_sem.at[b])
        pltpu.async_copy(lab_hbm.at[pl.ds(base, EC)], lab_v.at[b],
                         in_sem.at[b])

    def _wait_in(t, b):
        pltpu.make_async_copy(edge_hbm.at[pl.ds(0, 2), pl.ds(_base(t), EC)],
                              ei_v.at[b], in_sem.at[b]).wait()
        pltpu.make_async_copy(lab_hbm.at[pl.ds(_base(t), EC)], lab_v.at[b],
                              in_sem.at[b]).wait()

    def _issue_out(t, b):
        base = _base(t)
        pltpu.async_copy(mask_v.at[b], mask_hbm.at[pl.ds(base, EC)],
                         out_sem.at[b])
        pltpu.async_copy(gated_v.at[b], gated_hbm.at[pl.ds(base, EC)],
                         out_sem.at[b])

    def _wait_out(t, b):
        base = _base(t)
        pltpu.make_async_copy(mask_v.at[b], mask_hbm.at[pl.ds(base, EC)],
                              out_sem.at[b]).wait()
        pltpu.make_async_copy(gated_v.at[b], gated_hbm.at[pl.ds(base, EC)],
                              out_sem.at[b]).wait()

    def _compute(b):
        @plsc.parallel_loop(0, EC, step=L, unroll=GU)
        def _group(e):
            off = pl.multiple_of(e, 8)
            u = ei_v[b, 0, pl.ds(off, L)]
            v = ei_v[b, 1, pl.ds(off, L)]
            m = _bit(start_tbl, u) | _bit(start_tbl, v)
            labv = lab_v[b, pl.ds(off, L)]
            mask_v[b, pl.ds(off, L)] = m
            gated_v[b, pl.ds(off, L)] = jnp.where(
                m == 1, labv, jnp.zeros((L,), jnp.float32))

    def _guarded(t, fn):
        if t < CPW_MAX - 1:
            fn()
        else:
            pl.when(t < n_my)(fn)

    _guarded(0, lambda: _issue_in(0, 0))
    if CPW_MAX > 1:
        _guarded(1, lambda: _issue_in(1, 1))
    for t in range(CPW_MAX):
        b = t % 2

        def _phase(t=t, b=b):
            _wait_in(t, b)
            if t >= 2:
                _wait_out(t - 2, b)
            _compute(b)
            _issue_out(t, b)
            if t + 2 < CPW_MAX:
                _guarded(t + 2, lambda: _issue_in(t + 2, b))

        _guarded(t, _phase)
    for t in range(max(0, CPW_MAX - 3), CPW_MAX):
        def _drain(t=t):
            _wait_out(t, t % 2)
        pl.when(jnp.logical_and(t < n_my, t >= n_my - 2))(_drain)


@jax.jit
def _sc_call(edge_index, labels, start_locals, answer_locals):
    mesh = plsc.VectorSubcoreMesh(core_axis_name="c", subcore_axis_name="s",
                                  num_cores=NC, num_subcores=NS)
    out_type = (
        jax.ShapeDtypeStruct((N_EDGES,), jnp.int32),
        jax.ShapeDtypeStruct((N_EDGES,), jnp.float32),
        jax.ShapeDtypeStruct((N_NODES_PAD,), jnp.int32),
        jax.ShapeDtypeStruct((N_NODES_PAD,), jnp.int32),
    )
    scratch = [
        pltpu.VMEM((TBL_W,), jnp.int32),
        pltpu.VMEM((TBL_W,), jnp.int32),
        pltpu.VMEM((N_STARTS,), jnp.int32),
        pltpu.VMEM((N_ANSWERS,), jnp.int32),
        pltpu.VMEM((2, 2, EC), jnp.int32),
        pltpu.VMEM((2, EC), jnp.float32),
        pltpu.VMEM((2, EC), jnp.float32),
        pltpu.VMEM((2, EC), jnp.int32),
        pltpu.VMEM((2 * NODES_PW,), jnp.int32),
        pltpu.SemaphoreType.DMA((2,)),
        pltpu.SemaphoreType.DMA((2,)),
    ]
    params = pltpu.CompilerParams(needs_layout_passes=False)
    return pl.kernel(_sc_body, out_type=out_type, mesh=mesh,
                     scratch_types=scratch,
                     compiler_params=params)(edge_index, labels,
                                             start_locals, answer_locals)


def kernel(edge_index, edge_batch, node_global_ids, node_ptr, edge_ptr,
           start_node_locals, start_ptr, start_entity_ids, start_entity_ptr,
           answer_node_locals, answer_ptr, answer_entity_ids, edge_relations,
           edge_labels, top_edge_mask, gt_path_edge_local_ids, gt_edge_ptr,
           gt_path_exists, is_answer_reachable, bypass_action_mask):
    num_graphs = node_ptr.shape[0] - 1

    mask_i, gated_labels, ns_i, na_i = _sc_call(
        edge_index, edge_labels, start_node_locals, answer_node_locals)

    edge_starts_mask = mask_i.astype(bool)
    node_is_start = ns_i[:N_NODES].astype(bool)
    node_is_answer = na_i[:N_NODES].astype(bool)
    visited_nodes = node_is_start

    selected_mask = jnp.zeros((N_EDGES,), dtype=bool)
    selection_order = jnp.full((N_EDGES,), -1, dtype=jnp.int32)
    current_tail = jnp.full((num_graphs,), -1, dtype=jnp.int32)
    prev_tail = jnp.full((num_graphs,), -1, dtype=jnp.int32)
    done = jnp.zeros((num_graphs,), dtype=bool)
    step_counts = jnp.zeros((num_graphs,), dtype=jnp.int32)
    actions = jnp.full((num_graphs, MAX_STEPS + 1), -1, dtype=jnp.int32)
    answer_hits = jnp.zeros((num_graphs,), dtype=bool)
    start_counts = start_ptr[1:] - start_ptr[:-1]

    return (edge_starts_mask, node_is_start, node_is_answer, visited_nodes,
            selected_mask, selection_order, current_tail, prev_tail, done,
            step_counts, actions, answer_hits, start_counts, gated_labels)
